# baseline (speedup 1.0000x reference)
import functools

import jax
import jax.numpy as jnp
from jax import lax
from jax.experimental import pallas as pl
from jax.experimental.pallas import tpu as pltpu

_C_IN = 64
_C_OUT = 128
_EPS = 1e-5
_VMEM_LIMIT = 100 * 1024 * 1024


def _sel_kernel(x_ref, sel_ref, xs_ref, *, bn):
    cin, hw = x_ref.shape[1], x_ref.shape[2]
    p = sel_ref.shape[1]
    xm = x_ref[...].reshape(bn * cin, hw)
    xc = jnp.dot(xm, sel_ref[...], preferred_element_type=jnp.float32)
    xs_ref[...] = xc.astype(jnp.bfloat16).reshape(bn, cin, p)


def _moments_kernel(xs_ref, s1_ref, s2_ref, *, bn):
    i = pl.program_id(0)

    @pl.when(i == 0)
    def _():
        s1_ref[...] = jnp.zeros_like(s1_ref)
        s2_ref[...] = jnp.zeros_like(s2_ref)

    cin = xs_ref.shape[1]
    xb = xs_ref[...]
    s1 = s1_ref[...]
    s2 = s2_ref[...]
    for t in range(bn):
        xt = xb[t]
        s2 = s2 + lax.dot_general(
            xt, xt, (((1,), (1,)), ((), ())),
            preferred_element_type=jnp.float32)
        s1 = s1 + jnp.sum(xt.astype(jnp.float32), axis=1, keepdims=True)
    s1_ref[...] = s1
    s2_ref[...] = s2


def _apply_kernel(xs_ref, s1_ref, s2_ref, w_ref, g_ref, b_ref, o_ref,
                  ws_ref, bias_ref, *, bn, inv_count):
    i = pl.program_id(0)

    @pl.when(i == 0)
    def _():
        wm = w_ref[...]
        mean = (jnp.dot(wm, s1_ref[...], preferred_element_type=jnp.float32)
                * inv_count)
        ey2 = ((jnp.dot(wm, s2_ref[...], preferred_element_type=jnp.float32)
                * wm).sum(axis=1, keepdims=True) * inv_count)
        var = jnp.maximum(ey2 - mean * mean, 0.0)
        scale = g_ref[...] * lax.rsqrt(var + _EPS)
        ws_ref[...] = (wm * scale).astype(jnp.bfloat16)
        bias_ref[...] = b_ref[...] - mean * scale

    w = ws_ref[...]
    b = bias_ref[...]
    for t in range(bn):
        o_ref[t] = (
            jnp.dot(w, xs_ref[t], preferred_element_type=jnp.float32) + b)


def kernel(x_nchw, weight, gamma, beta):
    n, cin, h, w = x_nchw.shape
    assert cin == _C_IN and h % 2 == 0 and w % 2 == 0
    ho, wo = h // 2, w // 2
    p = ho * wo
    hw = h * w
    xb = x_nchw.astype(jnp.bfloat16).reshape(n, cin, hw)

    pos = jnp.arange(p, dtype=jnp.int32)
    src = (2 * w) * (pos // wo) + 2 * (pos % wo)
    sel = (jnp.arange(hw, dtype=jnp.int32)[:, None] == src[None, :]).astype(
        jnp.bfloat16)

    bn1 = 16
    xs = pl.pallas_call(
        functools.partial(_sel_kernel, bn=bn1),
        out_shape=jax.ShapeDtypeStruct((n, cin, p), jnp.bfloat16),
        grid=(n // bn1,),
        in_specs=[
            pl.BlockSpec((bn1, cin, hw), lambda i: (i, 0, 0)),
            pl.BlockSpec((hw, p), lambda i: (0, 0)),
        ],
        out_specs=pl.BlockSpec((bn1, cin, p), lambda i: (i, 0, 0)),
        compiler_params=pltpu.CompilerParams(
            dimension_semantics=("arbitrary",),
            vmem_limit_bytes=_VMEM_LIMIT,
        ),
    )(xb, sel)

    bnm = 8
    s1, s2 = pl.pallas_call(
        functools.partial(_moments_kernel, bn=bnm),
        out_shape=(
            jax.ShapeDtypeStruct((cin, 1), jnp.float32),
            jax.ShapeDtypeStruct((cin, cin), jnp.float32),
        ),
        grid=(n // bnm,),
        in_specs=[pl.BlockSpec((bnm, cin, p), lambda i: (i, 0, 0))],
        out_specs=(
            pl.BlockSpec((cin, 1), lambda i: (0, 0)),
            pl.BlockSpec((cin, cin), lambda i: (0, 0)),
        ),
        compiler_params=pltpu.CompilerParams(
            dimension_semantics=("arbitrary",),
            vmem_limit_bytes=_VMEM_LIMIT,
        ),
    )(xs)

    inv_count = 1.0 / float(n * p)
    wm = weight.reshape(_C_OUT, _C_IN).astype(jnp.float32)
    g2 = gamma.astype(jnp.float32).reshape(_C_OUT, 1)
    b2 = beta.astype(jnp.float32).reshape(_C_OUT, 1)
    bn2 = 4
    out = pl.pallas_call(
        functools.partial(_apply_kernel, bn=bn2, inv_count=inv_count),
        out_shape=jax.ShapeDtypeStruct((n, _C_OUT, p), jnp.float32),
        grid=(n // bn2,),
        in_specs=[
            pl.BlockSpec((bn2, cin, p), lambda i: (i, 0, 0)),
            pl.BlockSpec((cin, 1), lambda i: (0, 0)),
            pl.BlockSpec((cin, cin), lambda i: (0, 0)),
            pl.BlockSpec((_C_OUT, cin), lambda i: (0, 0)),
            pl.BlockSpec((_C_OUT, 1), lambda i: (0, 0)),
            pl.BlockSpec((_C_OUT, 1), lambda i: (0, 0)),
        ],
        out_specs=pl.BlockSpec((bn2, _C_OUT, p), lambda i: (i, 0, 0)),
        scratch_shapes=[
            pltpu.VMEM((_C_OUT, _C_IN), jnp.bfloat16),
            pltpu.VMEM((_C_OUT, 1), jnp.float32),
        ],
        compiler_params=pltpu.CompilerParams(
            dimension_semantics=("arbitrary",),
            vmem_limit_bytes=_VMEM_LIMIT,
        ),
    )(xs, s1, s2, wm, g2, b2)

    return out.reshape(n, _C_OUT, ho, wo)


# speedup vs baseline: 1.0673x; 1.0673x over previous
import functools

import jax
import jax.numpy as jnp
from jax import lax
from jax.experimental import pallas as pl
from jax.experimental.pallas import tpu as pltpu

_C_IN = 64
_C_OUT = 128
_EPS = 1e-5
_VMEM_LIMIT = 100 * 1024 * 1024


def _sel_moments_kernel(x_ref, sel_ref, xs_ref, s1_ref, s2_ref, *, bn):
    i = pl.program_id(0)

    @pl.when(i == 0)
    def _():
        s1_ref[...] = jnp.zeros_like(s1_ref)
        s2_ref[...] = jnp.zeros_like(s2_ref)

    cin, hw = x_ref.shape[1], x_ref.shape[2]
    p = sel_ref.shape[1]
    xm = x_ref[...].reshape(bn * cin, hw)
    xc = jnp.dot(xm, sel_ref[...], preferred_element_type=jnp.float32)
    xcb = xc.astype(jnp.bfloat16)
    xs_ref[...] = xcb.reshape(bn, cin, p)

    s1 = s1_ref[...]
    s2 = s2_ref[...]
    for t in range(bn):
        xt = xcb[t * cin:(t + 1) * cin]
        s2 = s2 + lax.dot_general(
            xt, xt, (((1,), (1,)), ((), ())),
            preferred_element_type=jnp.float32)
        s1 = s1 + jnp.sum(xt.astype(jnp.float32), axis=1, keepdims=True)
    s1_ref[...] = s1
    s2_ref[...] = s2


def _apply_kernel(xs_ref, s1_ref, s2_ref, w_ref, g_ref, b_ref, o_ref,
                  ws_ref, bias_ref, *, bn, inv_count):
    i = pl.program_id(0)

    @pl.when(i == 0)
    def _():
        wm = w_ref[...]
        mean = (jnp.dot(wm, s1_ref[...], preferred_element_type=jnp.float32)
                * inv_count)
        ey2 = ((jnp.dot(wm, s2_ref[...], preferred_element_type=jnp.float32)
                * wm).sum(axis=1, keepdims=True) * inv_count)
        var = jnp.maximum(ey2 - mean * mean, 0.0)
        scale = g_ref[...] * lax.rsqrt(var + _EPS)
        ws_ref[...] = (wm * scale).astype(jnp.bfloat16)
        bias_ref[...] = b_ref[...] - mean * scale

    w = ws_ref[...]
    b = bias_ref[...]
    for t in range(bn):
        o_ref[t] = (
            jnp.dot(w, xs_ref[t], preferred_element_type=jnp.float32) + b)


def kernel(x_nchw, weight, gamma, beta):
    n, cin, h, w = x_nchw.shape
    assert cin == _C_IN and h % 2 == 0 and w % 2 == 0
    ho, wo = h // 2, w // 2
    p = ho * wo
    hw = h * w
    xb = x_nchw.astype(jnp.bfloat16).reshape(n, cin, hw)

    pos = jnp.arange(p, dtype=jnp.int32)
    src = (2 * w) * (pos // wo) + 2 * (pos % wo)
    sel = (jnp.arange(hw, dtype=jnp.int32)[:, None] == src[None, :]).astype(
        jnp.bfloat16)

    bn1 = 16
    xs, s1, s2 = pl.pallas_call(
        functools.partial(_sel_moments_kernel, bn=bn1),
        out_shape=(
            jax.ShapeDtypeStruct((n, cin, p), jnp.bfloat16),
            jax.ShapeDtypeStruct((cin, 1), jnp.float32),
            jax.ShapeDtypeStruct((cin, cin), jnp.float32),
        ),
        grid=(n // bn1,),
        in_specs=[
            pl.BlockSpec((bn1, cin, hw), lambda i: (i, 0, 0)),
            pl.BlockSpec((hw, p), lambda i: (0, 0)),
        ],
        out_specs=(
            pl.BlockSpec((bn1, cin, p), lambda i: (i, 0, 0)),
            pl.BlockSpec((cin, 1), lambda i: (0, 0)),
            pl.BlockSpec((cin, cin), lambda i: (0, 0)),
        ),
        compiler_params=pltpu.CompilerParams(
            dimension_semantics=("arbitrary",),
            vmem_limit_bytes=_VMEM_LIMIT,
        ),
    )(xb, sel)

    inv_count = 1.0 / float(n * p)
    wm = weight.reshape(_C_OUT, _C_IN).astype(jnp.float32)
    g2 = gamma.astype(jnp.float32).reshape(_C_OUT, 1)
    b2 = beta.astype(jnp.float32).reshape(_C_OUT, 1)
    bn2 = 8
    out = pl.pallas_call(
        functools.partial(_apply_kernel, bn=bn2, inv_count=inv_count),
        out_shape=jax.ShapeDtypeStruct((n, _C_OUT, p), jnp.float32),
        grid=(n // bn2,),
        in_specs=[
            pl.BlockSpec((bn2, cin, p), lambda i: (i, 0, 0)),
            pl.BlockSpec((cin, 1), lambda i: (0, 0)),
            pl.BlockSpec((cin, cin), lambda i: (0, 0)),
            pl.BlockSpec((_C_OUT, cin), lambda i: (0, 0)),
            pl.BlockSpec((_C_OUT, 1), lambda i: (0, 0)),
            pl.BlockSpec((_C_OUT, 1), lambda i: (0, 0)),
        ],
        out_specs=pl.BlockSpec((bn2, _C_OUT, p), lambda i: (i, 0, 0)),
        scratch_shapes=[
            pltpu.VMEM((_C_OUT, _C_IN), jnp.bfloat16),
            pltpu.VMEM((_C_OUT, 1), jnp.float32),
        ],
        compiler_params=pltpu.CompilerParams(
            dimension_semantics=("arbitrary",),
            vmem_limit_bytes=_VMEM_LIMIT,
        ),
    )(xs, s1, s2, wm, g2, b2)

    return out.reshape(n, _C_OUT, ho, wo)


# speedup vs baseline: 1.0735x; 1.0057x over previous
import functools

import jax
import jax.numpy as jnp
from jax import lax
from jax.experimental import pallas as pl
from jax.experimental.pallas import tpu as pltpu

_C_IN = 64
_C_OUT = 128
_EPS = 1e-5
_VMEM_LIMIT = 100 * 1024 * 1024


def _sel_moments_kernel(x_ref, sel_ref, xs_ref, s1_ref, s2_ref, *, bn):
    i = pl.program_id(0)

    @pl.when(i == 0)
    def _():
        s1_ref[...] = jnp.zeros_like(s1_ref)
        s2_ref[...] = jnp.zeros_like(s2_ref)

    cin, hw = x_ref.shape[1], x_ref.shape[2]
    p = sel_ref.shape[1]
    xm = x_ref[...].reshape(bn * cin, hw)
    xc = jnp.dot(xm, sel_ref[...], preferred_element_type=jnp.float32)
    xcb = xc.astype(jnp.bfloat16)
    xs_ref[...] = xcb.reshape(bn, cin, p)

    s1 = s1_ref[...]
    s2 = s2_ref[...]
    for t in range(bn):
        xt = xcb[t * cin:(t + 1) * cin]
        s2 = s2 + lax.dot_general(
            xt, xt, (((1,), (1,)), ((), ())),
            preferred_element_type=jnp.float32)
        s1 = s1 + jnp.sum(xt.astype(jnp.float32), axis=1, keepdims=True)
    s1_ref[...] = s1
    s2_ref[...] = s2


def _apply_kernel(xs_ref, s1_ref, s2_ref, w_ref, g_ref, b_ref, o_ref,
                  ws_ref, bias_ref, *, bn, inv_count):
    i = pl.program_id(0)

    @pl.when(i == 0)
    def _():
        wm = w_ref[...]
        mean = (jnp.dot(wm, s1_ref[...], preferred_element_type=jnp.float32)
                * inv_count)
        ey2 = ((jnp.dot(wm, s2_ref[...], preferred_element_type=jnp.float32)
                * wm).sum(axis=1, keepdims=True) * inv_count)
        var = jnp.maximum(ey2 - mean * mean, 0.0)
        scale = g_ref[...] * lax.rsqrt(var + _EPS)
        ws_ref[...] = (wm * scale).astype(jnp.bfloat16)
        bias_ref[...] = b_ref[...] - mean * scale

    w = ws_ref[...]
    b = bias_ref[...]
    for t in range(bn):
        o_ref[t] = (
            jnp.dot(w, xs_ref[t], preferred_element_type=jnp.float32) + b)


def kernel(x_nchw, weight, gamma, beta):
    n, cin, h, w = x_nchw.shape
    assert cin == _C_IN and h % 2 == 0 and w % 2 == 0
    ho, wo = h // 2, w // 2
    p = ho * wo
    hw = h * w
    xb = x_nchw.astype(jnp.bfloat16).reshape(n, cin, hw)

    pos = jnp.arange(p, dtype=jnp.int32)
    src = (2 * w) * (pos // wo) + 2 * (pos % wo)
    sel = (jnp.arange(hw, dtype=jnp.int32)[:, None] == src[None, :]).astype(
        jnp.bfloat16)

    bn1 = 32
    xs, s1, s2 = pl.pallas_call(
        functools.partial(_sel_moments_kernel, bn=bn1),
        out_shape=(
            jax.ShapeDtypeStruct((n, cin, p), jnp.bfloat16),
            jax.ShapeDtypeStruct((cin, 1), jnp.float32),
            jax.ShapeDtypeStruct((cin, cin), jnp.float32),
        ),
        grid=(n // bn1,),
        in_specs=[
            pl.BlockSpec((bn1, cin, hw), lambda i: (i, 0, 0)),
            pl.BlockSpec((hw, p), lambda i: (0, 0)),
        ],
        out_specs=(
            pl.BlockSpec((bn1, cin, p), lambda i: (i, 0, 0)),
            pl.BlockSpec((cin, 1), lambda i: (0, 0)),
            pl.BlockSpec((cin, cin), lambda i: (0, 0)),
        ),
        compiler_params=pltpu.CompilerParams(
            dimension_semantics=("arbitrary",),
            vmem_limit_bytes=_VMEM_LIMIT,
        ),
    )(xb, sel)

    inv_count = 1.0 / float(n * p)
    wm = weight.reshape(_C_OUT, _C_IN).astype(jnp.float32)
    g2 = gamma.astype(jnp.float32).reshape(_C_OUT, 1)
    b2 = beta.astype(jnp.float32).reshape(_C_OUT, 1)
    bn2 = 16
    out = pl.pallas_call(
        functools.partial(_apply_kernel, bn=bn2, inv_count=inv_count),
        out_shape=jax.ShapeDtypeStruct((n, _C_OUT, p), jnp.float32),
        grid=(n // bn2,),
        in_specs=[
            pl.BlockSpec((bn2, cin, p), lambda i: (i, 0, 0)),
            pl.BlockSpec((cin, 1), lambda i: (0, 0)),
            pl.BlockSpec((cin, cin), lambda i: (0, 0)),
            pl.BlockSpec((_C_OUT, cin), lambda i: (0, 0)),
            pl.BlockSpec((_C_OUT, 1), lambda i: (0, 0)),
            pl.BlockSpec((_C_OUT, 1), lambda i: (0, 0)),
        ],
        out_specs=pl.BlockSpec((bn2, _C_OUT, p), lambda i: (i, 0, 0)),
        scratch_shapes=[
            pltpu.VMEM((_C_OUT, _C_IN), jnp.bfloat16),
            pltpu.VMEM((_C_OUT, 1), jnp.float32),
        ],
        compiler_params=pltpu.CompilerParams(
            dimension_semantics=("arbitrary",),
            vmem_limit_bytes=_VMEM_LIMIT,
        ),
    )(xs, s1, s2, wm, g2, b2)

    return out.reshape(n, _C_OUT, ho, wo)
